# Initial kernel scaffold; baseline (speedup 1.0000x reference)
import jax
import jax.numpy as jnp
from jax.experimental import pallas as pl


def kernel(ray_origins, ray_directions, coarse_weights, coarse_t_vals, u, num_fine_samples):
    raise NotImplementedError("write your pallas kernel here")


# speedup vs baseline: 4.0131x; 4.0131x over previous
import functools

import jax
import jax.numpy as jnp
from jax.experimental import pallas as pl

B = 16384
NC = 64
NF = 128
NT = NC + NF


def _body(o_ref, d_ref, w_ref, t_ref, u_ref, tout_ref, pout_ref):
    f32 = jnp.float32
    i32 = jnp.int32
    R = w_ref.shape[0]

    w = w_ref[...] + 1e-5
    s = jnp.sum(w, axis=-1, keepdims=True)
    pdf = w / s
    c = pdf
    sh = 1
    while sh < NC:
        c = c + jnp.concatenate(
            [jnp.zeros((R, sh), f32), c[:, : NC - sh]], axis=-1)
        sh *= 2
    cdf_excl = c - pdf

    u = u_ref[...]
    tv = t_ref[...]

    le = (c[:, None, :] <= u[:, :, None]).astype(i32)
    m = jnp.sum(le, axis=-1)
    below = jnp.minimum(m, NC - 1)
    above = jnp.minimum(m + 1, NC - 1)

    k_iota = jax.lax.broadcasted_iota(i32, (R, NF, NC), 2)
    ohb = k_iota == below[:, :, None]
    oha = k_iota == above[:, :, None]
    tv3 = tv[:, None, :]
    ce3 = cdf_excl[:, None, :]
    t_below = jnp.sum(jnp.where(ohb, tv3, 0.0), axis=-1)
    t_above = jnp.sum(jnp.where(oha, tv3, 0.0), axis=-1)
    c_below = jnp.sum(jnp.where(ohb, ce3, 0.0), axis=-1)
    c_above = jnp.sum(jnp.where(oha, ce3, 0.0), axis=-1)

    denom = c_above - c_below
    denom = jnp.where(denom < 1e-5, 1.0, denom)
    frac = (u - c_below) / denom
    fine = t_below + frac * (t_above - t_below)

    i_iota = jax.lax.broadcasted_iota(i32, (R, NF, NF), 1)
    j_iota = jax.lax.broadcasted_iota(i32, (R, NF, NF), 2)
    ui = u[:, :, None]
    uj = u[:, None, :]
    rank_u = jnp.sum(
        ((uj < ui) | ((uj == ui) & (j_iota < i_iota))).astype(i32), axis=-1)

    cnt_c_le = jnp.sum((tv[:, None, :] <= fine[:, :, None]).astype(i32),
                       axis=-1)
    pos_f = rank_u + cnt_c_le
    cnt_f_lt = jnp.sum((fine[:, None, :] < tv[:, :, None]).astype(i32),
                       axis=-1)
    pos_c = jax.lax.broadcasted_iota(i32, (R, NC), 1) + cnt_f_lt

    out_iota_f = jax.lax.broadcasted_iota(i32, (R, NF, NT), 2)
    out_iota_c = jax.lax.broadcasted_iota(i32, (R, NC, NT), 2)
    contrib_f = jnp.sum(
        jnp.where(out_iota_f == pos_f[:, :, None], fine[:, :, None], 0.0),
        axis=1)
    contrib_c = jnp.sum(
        jnp.where(out_iota_c == pos_c[:, :, None], tv[:, :, None], 0.0),
        axis=1)
    t_out = contrib_f + contrib_c

    tout_ref[...] = t_out
    o = o_ref[...]
    d = d_ref[...]
    pout_ref[...] = o[:, :, None] + d[:, :, None] * t_out[:, None, :]


@functools.partial(jax.jit, static_argnames=())
def _run(ray_origins, ray_directions, coarse_weights, coarse_t_vals, u):
    R = 16
    grid = (B // R,)
    t_out, p_out = pl.pallas_call(
        _body,
        grid=grid,
        in_specs=[
            pl.BlockSpec((R, 3), lambda b: (b, 0)),
            pl.BlockSpec((R, 3), lambda b: (b, 0)),
            pl.BlockSpec((R, NC), lambda b: (b, 0)),
            pl.BlockSpec((R, NC), lambda b: (b, 0)),
            pl.BlockSpec((R, NF), lambda b: (b, 0)),
        ],
        out_specs=[
            pl.BlockSpec((R, NT), lambda b: (b, 0)),
            pl.BlockSpec((R, 3, NT), lambda b: (b, 0, 0)),
        ],
        out_shape=[
            jax.ShapeDtypeStruct((B, NT), jnp.float32),
            jax.ShapeDtypeStruct((B, 3, NT), jnp.float32),
        ],
    )(ray_origins, ray_directions, coarse_weights, coarse_t_vals, u)
    return t_out, p_out


def kernel(ray_origins, ray_directions, coarse_weights, coarse_t_vals, u,
           num_fine_samples):
    t_out, p_out = _run(ray_origins, ray_directions, coarse_weights,
                        coarse_t_vals, u)
    fine_points = jnp.transpose(p_out, (0, 2, 1))
    return (fine_points, t_out)


# speedup vs baseline: 171.7465x; 42.7966x over previous
import functools

import jax
import jax.numpy as jnp
from jax.experimental import pallas as pl

B = 16384
NC = 64
NF = 128
NT = NC + NF
NM = 256


def _iota(R, n):
    return jax.lax.broadcasted_iota(jnp.int32, (R, n), 1)


def _bitonic_sort_desc(x):
    R, n = x.shape
    idx = _iota(R, n)
    k = 2
    while k <= n:
        j = k // 2
        while j >= 1:
            upper = (idx & j) != 0
            px = jnp.where(upper, jnp.roll(x, j, axis=-1),
                           jnp.roll(x, -j, axis=-1))
            mn = jnp.minimum(x, px)
            mx = jnp.maximum(x, px)
            asc = (idx & k) != 0
            take_min = asc != upper
            x = jnp.where(take_min, mn, mx)
            j //= 2
        k *= 2
    return x


def _merge_asc(x, chans):
    R, n = x.shape
    idx = _iota(R, n)
    j = n // 2
    while j >= 1:
        upper = (idx & j) != 0
        newx = jnp.where(upper,
                         jnp.maximum(x, jnp.roll(x, j, axis=-1)),
                         jnp.minimum(x, jnp.roll(x, -j, axis=-1)))
        own_wins = newx == x
        new_chans = []
        for ch in chans:
            pch = jnp.where(upper, jnp.roll(ch, j, axis=-1),
                            jnp.roll(ch, -j, axis=-1))
            new_chans.append(jnp.where(own_wins, ch, pch))
        chans = new_chans
        x = newx
        j //= 2
    return x, chans


def _prefix_max(y, n):
    R = y.shape[0]
    idx = _iota(R, n)
    d = 1
    while d < n:
        sh = jnp.where(idx < d, 0.0, jnp.roll(y, d, axis=-1))
        y = jnp.maximum(y, sh)
        d *= 2
    return y


def _suffix_min(y, n):
    R = y.shape[0]
    idx = _iota(R, n)
    inf = jnp.float32(jnp.inf)
    d = 1
    while d < n:
        sh = jnp.where(idx >= n - d, inf, jnp.roll(y, -d, axis=-1))
        y = jnp.minimum(y, sh)
        d *= 2
    return y


def _body(o_ref, d_ref, w_ref, t_ref, u_ref, tout_ref, pout_ref):
    f32 = jnp.float32
    R = w_ref.shape[0]
    inf = jnp.float32(jnp.inf)

    w = w_ref[...] + 1e-5
    s = jnp.sum(w, axis=-1, keepdims=True)
    pdf = w / s
    c = pdf
    sh = 1
    while sh < NC:
        c = c + jnp.concatenate(
            [jnp.zeros((R, sh), f32), c[:, : NC - sh]], axis=-1)
        sh *= 2
    keys_c = c - pdf

    tv = t_ref[...]
    u_desc = _bitonic_sort_desc(u_ref[...])

    pad = jnp.full((R, NM - NC - NF), inf, f32)
    x = jnp.concatenate([keys_c, pad, u_desc], axis=-1)
    tvch = jnp.concatenate([tv, jnp.full((R, NM - NC), -1.0, f32)], axis=-1)

    x, (tvch,) = _merge_asc(x, (tvch,))

    key = x[:, :NT]
    tvch = tvch[:, :NT]
    is_c = tvch > 0.0

    c_bel = _prefix_max(jnp.where(is_c, key, 0.0), NT)
    t_bel = _prefix_max(jnp.maximum(tvch, 0.0), NT)
    ca = _suffix_min(jnp.where(is_c, key, inf), NT)
    ta = _suffix_min(jnp.where(is_c, tvch, inf), NT)
    c_abv = jnp.where(ca == inf, c_bel, ca)
    t_abv = jnp.where(ta == inf, t_bel, ta)
    t_bel = jnp.where(t_bel > 0.0, t_bel, t_abv)

    denom = c_abv - c_bel
    denom = jnp.where(denom < 1e-5, 1.0, denom)
    frac = (key - c_bel) / denom
    fine = t_bel + frac * (t_abv - t_bel)
    t_out = jnp.where(is_c, tvch, fine)

    tout_ref[...] = t_out
    o = o_ref[...]
    d = d_ref[...]
    pout_ref[...] = o[:, :, None] + d[:, :, None] * t_out[:, None, :]


@functools.partial(jax.jit, static_argnames=())
def _run(ray_origins, ray_directions, coarse_weights, coarse_t_vals, u):
    R = 512
    grid = (B // R,)
    t_out, p_out = pl.pallas_call(
        _body,
        grid=grid,
        in_specs=[
            pl.BlockSpec((R, 3), lambda b: (b, 0)),
            pl.BlockSpec((R, 3), lambda b: (b, 0)),
            pl.BlockSpec((R, NC), lambda b: (b, 0)),
            pl.BlockSpec((R, NC), lambda b: (b, 0)),
            pl.BlockSpec((R, NF), lambda b: (b, 0)),
        ],
        out_specs=[
            pl.BlockSpec((R, NT), lambda b: (b, 0)),
            pl.BlockSpec((R, 3, NT), lambda b: (b, 0, 0)),
        ],
        out_shape=[
            jax.ShapeDtypeStruct((B, NT), jnp.float32),
            jax.ShapeDtypeStruct((B, 3, NT), jnp.float32),
        ],
    )(ray_origins, ray_directions, coarse_weights, coarse_t_vals, u)
    return t_out, p_out


def kernel(ray_origins, ray_directions, coarse_weights, coarse_t_vals, u,
           num_fine_samples):
    t_out, p_out = _run(ray_origins, ray_directions, coarse_weights,
                        coarse_t_vals, u)
    fine_points = jnp.transpose(p_out, (0, 2, 1))
    return (fine_points, t_out)


# speedup vs baseline: 177.5660x; 1.0339x over previous
import functools

import jax
import jax.numpy as jnp
from jax.experimental import pallas as pl

B = 16384
NC = 64
NF = 128
NT = NC + NF
NM = 256


def _iota(R, n):
    return jax.lax.broadcasted_iota(jnp.int32, (R, n), 1)


def _bitonic_sort_desc(x):
    R, n = x.shape
    idx = _iota(R, n)
    k = 2
    while k <= n:
        j = k // 2
        while j >= 1:
            upper = (idx & j) != 0
            px = jnp.where(upper, jnp.roll(x, j, axis=-1),
                           jnp.roll(x, -j, axis=-1))
            mn = jnp.minimum(x, px)
            mx = jnp.maximum(x, px)
            asc = (idx & k) != 0
            take_min = asc != upper
            x = jnp.where(take_min, mn, mx)
            j //= 2
        k *= 2
    return x


def _merge_asc(x, chans):
    R, n = x.shape
    idx = _iota(R, n)
    j = n // 2
    while j >= 1:
        upper = (idx & j) != 0
        newx = jnp.where(upper,
                         jnp.maximum(x, jnp.roll(x, j, axis=-1)),
                         jnp.minimum(x, jnp.roll(x, -j, axis=-1)))
        own_wins = newx == x
        new_chans = []
        for ch in chans:
            pch = jnp.where(upper, jnp.roll(ch, j, axis=-1),
                            jnp.roll(ch, -j, axis=-1))
            new_chans.append(jnp.where(own_wins, ch, pch))
        chans = new_chans
        x = newx
        j //= 2
    return x, chans


def _prefix_max(y, n):
    R = y.shape[0]
    idx = _iota(R, n)
    d = 1
    while d < n:
        sh = jnp.where(idx < d, 0.0, jnp.roll(y, d, axis=-1))
        y = jnp.maximum(y, sh)
        d *= 2
    return y


def _suffix_min(y, n):
    R = y.shape[0]
    idx = _iota(R, n)
    inf = jnp.float32(jnp.inf)
    d = 1
    while d < n:
        sh = jnp.where(idx >= n - d, inf, jnp.roll(y, -d, axis=-1))
        y = jnp.minimum(y, sh)
        d *= 2
    return y


def _body(o_ref, d_ref, w_ref, t_ref, u_ref, tout_ref, pout_ref):
    f32 = jnp.float32
    R = w_ref.shape[0]
    inf = jnp.float32(jnp.inf)

    w = w_ref[...] + 1e-5
    s = jnp.sum(w, axis=-1, keepdims=True)
    pdf = w / s
    c = pdf
    sh = 1
    while sh < NC:
        c = c + jnp.concatenate(
            [jnp.zeros((R, sh), f32), c[:, : NC - sh]], axis=-1)
        sh *= 2
    keys_c = c - pdf

    tv = t_ref[...]
    u_desc = _bitonic_sort_desc(u_ref[...])

    pad = jnp.full((R, NM - NC - NF), inf, f32)
    x = jnp.concatenate([keys_c, pad, u_desc], axis=-1)
    tvch = jnp.concatenate([tv, jnp.full((R, NM - NC), -1.0, f32)], axis=-1)

    x, (tvch,) = _merge_asc(x, (tvch,))

    key = x[:, :NT]
    tvch = tvch[:, :NT]
    is_c = tvch > 0.0

    c_bel = _prefix_max(jnp.where(is_c, key, 0.0), NT)
    t_bel = _prefix_max(jnp.maximum(tvch, 0.0), NT)
    ca = _suffix_min(jnp.where(is_c, key, inf), NT)
    ta = _suffix_min(jnp.where(is_c, tvch, inf), NT)
    c_abv = jnp.where(ca == inf, c_bel, ca)
    t_abv = jnp.where(ta == inf, t_bel, ta)
    t_bel = jnp.where(t_bel > 0.0, t_bel, t_abv)

    denom = c_abv - c_bel
    denom = jnp.where(denom < 1e-5, 1.0, denom)
    frac = (key - c_bel) / denom
    fine = t_bel + frac * (t_abv - t_bel)
    t_out = jnp.where(is_c, tvch, fine)

    tout_ref[...] = t_out
    o = o_ref[...]
    d = d_ref[...]
    oT = jnp.transpose(o, (1, 0))
    dT = jnp.transpose(d, (1, 0))
    pout_ref[...] = oT[:, :, None] + dT[:, :, None] * t_out[None, :, :]


@functools.partial(jax.jit, static_argnames=())
def _run(ray_origins, ray_directions, coarse_weights, coarse_t_vals, u):
    R = 512
    grid = (B // R,)
    t_out, p_out = pl.pallas_call(
        _body,
        grid=grid,
        in_specs=[
            pl.BlockSpec((R, 3), lambda b: (b, 0)),
            pl.BlockSpec((R, 3), lambda b: (b, 0)),
            pl.BlockSpec((R, NC), lambda b: (b, 0)),
            pl.BlockSpec((R, NC), lambda b: (b, 0)),
            pl.BlockSpec((R, NF), lambda b: (b, 0)),
        ],
        out_specs=[
            pl.BlockSpec((R, NT), lambda b: (b, 0)),
            pl.BlockSpec((3, R, NT), lambda b: (0, b, 0)),
        ],
        out_shape=[
            jax.ShapeDtypeStruct((B, NT), jnp.float32),
            jax.ShapeDtypeStruct((3, B, NT), jnp.float32),
        ],
    )(ray_origins, ray_directions, coarse_weights, coarse_t_vals, u)
    return t_out, p_out


def kernel(ray_origins, ray_directions, coarse_weights, coarse_t_vals, u,
           num_fine_samples):
    t_out, p_out = _run(ray_origins, ray_directions, coarse_weights,
                        coarse_t_vals, u)
    fine_points = jnp.transpose(p_out, (1, 2, 0))
    return (fine_points, t_out)
